# speedup vs baseline: 5.1767x; 5.1767x over previous
import functools

import jax
import jax.numpy as jnp
from jax import lax
from jax.experimental import pallas as pl
from jax.experimental.pallas import tpu as pltpu
from jax.experimental.pallas import tpu_sc as plsc

B, L, DN, D = 4096, 200, 128, 128
V_ITEM, V_CATE = 100002, 1002
BL = B * L
NC, NS = 2, 16
NW = NC * NS
PER_W = BL // NW
CH = 128
NCH = PER_W // CH

_mesh = plsc.VectorSubcoreMesh(core_axis_name="c", subcore_axis_name="s")


@functools.partial(
    pl.kernel,
    out_type=jax.ShapeDtypeStruct((BL, D), jnp.float32),
    mesh=_mesh,
    scratch_types=[
        pltpu.VMEM((NCH, CH), jnp.int32),
        pltpu.VMEM((NCH, CH), jnp.int32),
        pltpu.VMEM((CH, D), jnp.float32),
        pltpu.VMEM((CH, D), jnp.float32),
        pltpu.VMEM_SHARED((V_CATE, D), jnp.float32),
        pltpu.SemaphoreType.DMA,
        pltpu.SemaphoreType.DMA,
    ],
)
def _sc_gather_sum(item_id_hbm, cate_id_hbm, item_tab_hbm, cate_tab_hbm,
                   out_hbm, idx_i, idx_c, rows_i, rows_c, cate_sh,
                   sem_i, sem_c):
    c = lax.axis_index("c")
    s = lax.axis_index("s")
    wid = s * NC + c

    @pl.when(s == 0)
    def _():
        pltpu.sync_copy(cate_tab_hbm, cate_sh)
    plsc.subcore_barrier()

    pltpu.sync_copy(item_id_hbm.at[wid], idx_i)
    pltpu.sync_copy(cate_id_hbm.at[wid], idx_c)

    base = wid * PER_W

    def chunk(j, carry):
        gi = pltpu.async_copy(item_tab_hbm.at[idx_i.at[j]], rows_i, sem_i)
        gc = pltpu.async_copy(cate_sh.at[idx_c.at[j]], rows_c, sem_c)
        gi.wait()
        gc.wait()

        def add_row(r, carry2):
            for q in range(D // 16):
                sl = pl.ds(q * 16, 16)
                plsc.addupdate(rows_i.at[r, sl], rows_c[r, sl])
            return carry2

        lax.fori_loop(0, CH, add_row, 0)
        pltpu.sync_copy(rows_i, out_hbm.at[pl.ds(base + j * CH, CH)])
        return carry

    lax.fori_loop(0, NCH, chunk, 0)


BB = 8
_GRID = B // BB


def _tc_body(gath_ref, price_ref, w_ref, b_ref, pos_ref, len_ref,
             out_ref, mask_ref):
    x = price_ref[...].reshape(BB * L, DN)
    y = jnp.dot(x, w_ref[...], preferred_element_type=jnp.float32)
    y = y.reshape(BB, L, D)
    out_ref[...] = y + gath_ref[...] + pos_ref[...] + b_ref[...].reshape(1, 1, D)
    lens = len_ref[...]
    mask_ref[...] = lax.broadcasted_iota(jnp.int32, (BB, L), 1) < lens


def kernel(item_id, cate_id, price, length, item_table, cate_table, W, b,
           pos_table):
    ids_i = item_id.astype(jnp.int32).reshape(NW, NCH, CH)
    ids_c = cate_id.astype(jnp.int32).reshape(NW, NCH, CH)
    gathered = _sc_gather_sum(ids_i, ids_c, item_table, cate_table)
    gathered = gathered.reshape(B, L, D)

    pos = pos_table[:L][None]
    b2 = b.reshape(1, D)
    seq, mask = pl.pallas_call(
        _tc_body,
        grid=(_GRID,),
        in_specs=[
            pl.BlockSpec((BB, L, D), lambda i: (i, 0, 0)),
            pl.BlockSpec((BB, L, DN), lambda i: (i, 0, 0)),
            pl.BlockSpec((DN, D), lambda i: (0, 0)),
            pl.BlockSpec((1, D), lambda i: (0, 0)),
            pl.BlockSpec((1, L, D), lambda i: (0, 0, 0)),
            pl.BlockSpec((BB, 1), lambda i: (i, 0)),
        ],
        out_specs=[
            pl.BlockSpec((BB, L, D), lambda i: (i, 0, 0)),
            pl.BlockSpec((BB, L), lambda i: (i, 0)),
        ],
        out_shape=[
            jax.ShapeDtypeStruct((B, L, D), jnp.float32),
            jax.ShapeDtypeStruct((B, L), jnp.bool_),
        ],
    )(gathered, price, W, b2, pos, length)
    return seq, mask


# speedup vs baseline: 7.2824x; 1.4068x over previous
import functools

import jax
import jax.numpy as jnp
from jax import lax
from jax.experimental import pallas as pl
from jax.experimental.pallas import tpu as pltpu
from jax.experimental.pallas import tpu_sc as plsc

B, L, DN, D = 4096, 200, 128, 128
V_ITEM, V_CATE = 100002, 1002
DW = D // 2
BL = B * L
NC, NS = 2, 16
NW = NC * NS
PER_W = BL // NW
CH = 64
NCH = PER_W // CH

_mesh = plsc.VectorSubcoreMesh(core_axis_name="c", subcore_axis_name="s")


@functools.partial(
    pl.kernel,
    out_type=jax.ShapeDtypeStruct((BL, DW), jnp.int32),
    mesh=_mesh,
    compiler_params=pltpu.CompilerParams(needs_layout_passes=False),
    scratch_types=[
        pltpu.VMEM((2, CH), jnp.int32),
        pltpu.VMEM((2, CH), jnp.int32),
        pltpu.VMEM((2, CH, D), jnp.float32),
        pltpu.VMEM((2, CH, D), jnp.float32),
        pltpu.VMEM((CH, DW), jnp.int32),
        pltpu.VMEM_SHARED((V_CATE, D), jnp.float32),
        pltpu.SemaphoreType.DMA,
        pltpu.SemaphoreType.DMA,
        pltpu.SemaphoreType.DMA,
        pltpu.SemaphoreType.DMA,
        pltpu.SemaphoreType.DMA,
        pltpu.SemaphoreType.DMA,
    ],
)
def _sc_gather_sum(item_id_hbm, cate_id_hbm, item_tab_hbm, cate_tab_hbm,
                   out_hbm, idx_i, idx_c, rows_i, rows_c, out_st, cate_sh,
                   sem_i0, sem_i1, sem_c0, sem_c1, sem_x0, sem_x1):
    c = lax.axis_index("c")
    s = lax.axis_index("s")
    wid = s * NC + c
    sem_i = (sem_i0, sem_i1)
    sem_c = (sem_c0, sem_c1)
    sem_x = (sem_x0, sem_x1)

    @pl.when(s == 0)
    def _():
        pltpu.sync_copy(cate_tab_hbm, cate_sh)
    plsc.subcore_barrier()

    base = wid * PER_W

    def issue_idx(j, b):
        pltpu.async_copy(item_id_hbm.at[wid, j], idx_i.at[b], sem_x[b])
        pltpu.async_copy(cate_id_hbm.at[wid, j], idx_c.at[b], sem_x[b])

    def wait_idx(j, b):
        pltpu.make_async_copy(item_id_hbm.at[wid, j], idx_i.at[b],
                              sem_x[b]).wait()
        pltpu.make_async_copy(cate_id_hbm.at[wid, j], idx_c.at[b],
                              sem_x[b]).wait()

    def issue_gather(b):
        pltpu.async_copy(item_tab_hbm.at[idx_i.at[b]], rows_i.at[b], sem_i[b])
        pltpu.async_copy(cate_sh.at[idx_c.at[b]], rows_c.at[b], sem_c[b])

    def wait_gather(b):
        pltpu.make_async_copy(item_tab_hbm.at[idx_i.at[b]], rows_i.at[b],
                              sem_i[b]).wait()
        pltpu.make_async_copy(cate_sh.at[idx_c.at[b]], rows_c.at[b],
                              sem_c[b]).wait()

    def process(j, b):
        def add_row(r, carry):
            for q in range(DW // 16):
                lo = pl.ds(q * 16, 16)
                hi = pl.ds(DW + q * 16, 16)
                a = rows_i[b, r, lo] + rows_c[b, r, lo]
                z = rows_i[b, r, hi] + rows_c[b, r, hi]
                w = plsc.pack(a, z, format=plsc.PackFormat.INTERLEAVED)
                out_st[r, lo] = plsc.bitcast(w, jnp.int32)
            return carry

        lax.fori_loop(0, CH, add_row, 0)
        pltpu.sync_copy(out_st, out_hbm.at[pl.ds(base + j * CH, CH)])

    def half(i, par):
        b = par
        j = 2 * i + par
        wait_idx(j + 1, 1 - b)
        issue_gather(1 - b)
        wait_gather(b)

        @pl.when(j + 2 < NCH)
        def _():
            issue_idx(j + 2, b)

        process(j, b)

    issue_idx(0, 0)
    wait_idx(0, 0)
    issue_gather(0)
    issue_idx(1, 1)

    def body(i, carry):
        half(i, 0)
        half(i, 1)
        return carry

    lax.fori_loop(0, NCH // 2 - 1, body, 0)
    jlast = NCH - 2
    wait_idx(jlast + 1, 1)
    issue_gather(1)
    wait_gather(0)
    process(jlast, 0)
    wait_gather(1)
    process(jlast + 1, 1)


BB = 16
_GRID = B // BB


def _tc_body(gath_ref, price_ref, w_ref, b_ref, pos_ref, len_ref,
             out_ref, mask_ref):
    x = price_ref[...].reshape(BB * L, DN)
    y = jnp.dot(x, w_ref[...], preferred_element_type=jnp.float32)
    y = y.reshape(BB, L, D)
    wu = lax.bitcast_convert_type(gath_ref[...], jnp.uint32)
    g_lo = lax.bitcast_convert_type(wu << 16, jnp.float32)
    g_hi = lax.bitcast_convert_type(wu & jnp.uint32(0xFFFF0000), jnp.float32)
    g = jnp.concatenate([g_lo, g_hi], axis=-1)
    out_ref[...] = y + g + pos_ref[...] + b_ref[...].reshape(1, 1, D)
    lens = len_ref[...]
    mask_ref[...] = lax.broadcasted_iota(jnp.int32, (BB, L), 1) < lens


def kernel(item_id, cate_id, price, length, item_table, cate_table, W, b,
           pos_table):
    ids_i = item_id.astype(jnp.int32).reshape(NW, NCH, CH)
    ids_c = cate_id.astype(jnp.int32).reshape(NW, NCH, CH)
    gathered = _sc_gather_sum(ids_i, ids_c, item_table, cate_table)
    gathered = gathered.reshape(B, L, DW)

    pos = pos_table[:L][None]
    b2 = b.reshape(1, D)
    seq, mask = pl.pallas_call(
        _tc_body,
        grid=(_GRID,),
        in_specs=[
            pl.BlockSpec((BB, L, DW), lambda i: (i, 0, 0)),
            pl.BlockSpec((BB, L, DN), lambda i: (i, 0, 0)),
            pl.BlockSpec((DN, D), lambda i: (0, 0)),
            pl.BlockSpec((1, D), lambda i: (0, 0)),
            pl.BlockSpec((1, L, D), lambda i: (0, 0, 0)),
            pl.BlockSpec((BB, 1), lambda i: (i, 0)),
        ],
        out_specs=[
            pl.BlockSpec((BB, L, D), lambda i: (i, 0, 0)),
            pl.BlockSpec((BB, L), lambda i: (i, 0)),
        ],
        out_shape=[
            jax.ShapeDtypeStruct((B, L, D), jnp.float32),
            jax.ShapeDtypeStruct((B, L), jnp.bool_),
        ],
    )(gathered, price, W, b2, pos, length)
    return seq, mask


# speedup vs baseline: 7.6259x; 1.0472x over previous
import functools

import jax
import jax.numpy as jnp
from jax import lax
from jax.experimental import pallas as pl
from jax.experimental.pallas import tpu as pltpu
from jax.experimental.pallas import tpu_sc as plsc

B, L, DN, D = 4096, 200, 128, 128
V_ITEM, V_CATE = 100002, 1002
DW = D // 2
BL = B * L
NC, NS = 2, 16
NW = NC * NS
PER_W = BL // NW
CH = 64
NCH = PER_W // CH

_mesh = plsc.VectorSubcoreMesh(core_axis_name="c", subcore_axis_name="s")


@functools.partial(
    pl.kernel,
    out_type=jax.ShapeDtypeStruct((BL, DW), jnp.int32),
    mesh=_mesh,
    compiler_params=pltpu.CompilerParams(needs_layout_passes=False),
    scratch_types=[
        pltpu.VMEM((2, CH), jnp.int32),
        pltpu.VMEM((2, CH), jnp.int32),
        pltpu.VMEM((2, CH, D), jnp.float32),
        pltpu.VMEM((2, CH, D), jnp.float32),
        pltpu.VMEM((CH, DW), jnp.int32),
        pltpu.VMEM_SHARED((V_CATE, D), jnp.float32),
        pltpu.SemaphoreType.DMA,
        pltpu.SemaphoreType.DMA,
        pltpu.SemaphoreType.DMA,
        pltpu.SemaphoreType.DMA,
        pltpu.SemaphoreType.DMA,
        pltpu.SemaphoreType.DMA,
    ],
)
def _sc_gather_sum(item_id_hbm, cate_id_hbm, item_tab_hbm, cate_tab_hbm,
                   out_hbm, idx_i, idx_c, rows_i, rows_c, out_st, cate_sh,
                   sem_i0, sem_i1, sem_c0, sem_c1, sem_x0, sem_x1):
    c = lax.axis_index("c")
    s = lax.axis_index("s")
    wid = s * NC + c
    sem_i = (sem_i0, sem_i1)
    sem_c = (sem_c0, sem_c1)
    sem_x = (sem_x0, sem_x1)

    @pl.when(s == 0)
    def _():
        pltpu.sync_copy(cate_tab_hbm, cate_sh)
    plsc.subcore_barrier()

    base = wid * PER_W

    def issue_idx(j, b):
        pltpu.async_copy(item_id_hbm.at[wid, j], idx_i.at[b], sem_x[b])
        pltpu.async_copy(cate_id_hbm.at[wid, j], idx_c.at[b], sem_x[b])

    def wait_idx(j, b):
        pltpu.make_async_copy(item_id_hbm.at[wid, j], idx_i.at[b],
                              sem_x[b]).wait()
        pltpu.make_async_copy(cate_id_hbm.at[wid, j], idx_c.at[b],
                              sem_x[b]).wait()

    def issue_gather(b):
        pltpu.async_copy(item_tab_hbm.at[idx_i.at[b]], rows_i.at[b], sem_i[b])
        pltpu.async_copy(cate_sh.at[idx_c.at[b]], rows_c.at[b], sem_c[b])

    def wait_gather(b):
        pltpu.make_async_copy(item_tab_hbm.at[idx_i.at[b]], rows_i.at[b],
                              sem_i[b]).wait()
        pltpu.make_async_copy(cate_sh.at[idx_c.at[b]], rows_c.at[b],
                              sem_c[b]).wait()

    def process(j, b):
        def add_row(r, carry):
            for q in range(DW // 16):
                lo = pl.ds(q * 16, 16)
                hi = pl.ds(DW + q * 16, 16)
                a = rows_i[b, r, lo] + rows_c[b, r, lo]
                z = rows_i[b, r, hi] + rows_c[b, r, hi]
                w = plsc.pack(a, z, format=plsc.PackFormat.INTERLEAVED)
                out_st[r, lo] = plsc.bitcast(w, jnp.int32)
            return carry

        lax.fori_loop(0, CH, add_row, 0)
        pltpu.sync_copy(out_st, out_hbm.at[pl.ds(base + j * CH, CH)])

    def half(i, par):
        b = par
        j = 2 * i + par
        wait_idx(j + 1, 1 - b)
        issue_gather(1 - b)
        wait_gather(b)

        @pl.when(j + 2 < NCH)
        def _():
            issue_idx(j + 2, b)

        process(j, b)

    issue_idx(0, 0)
    wait_idx(0, 0)
    issue_gather(0)
    issue_idx(1, 1)

    def body(i, carry):
        half(i, 0)
        half(i, 1)
        return carry

    lax.fori_loop(0, NCH // 2 - 1, body, 0)
    jlast = NCH - 2
    wait_idx(jlast + 1, 1)
    issue_gather(1)
    wait_gather(0)
    process(jlast, 0)
    wait_gather(1)
    process(jlast + 1, 1)


BB = 32
_GRID = B // BB


def _tc_body(gath_ref, price_ref, w_ref, b_ref, pos_ref, len_ref,
             out_ref, mask_ref):
    x = price_ref[...].reshape(BB * L, DN)
    y = jnp.dot(x, w_ref[...], preferred_element_type=jnp.float32)
    y = y.reshape(BB, L, D)
    wu = lax.bitcast_convert_type(gath_ref[...], jnp.uint32)
    g_lo = lax.bitcast_convert_type(wu << 16, jnp.float32)
    g_hi = lax.bitcast_convert_type(wu & jnp.uint32(0xFFFF0000), jnp.float32)
    g = jnp.concatenate([g_lo, g_hi], axis=-1)
    out_ref[...] = y + g + pos_ref[...] + b_ref[...].reshape(1, 1, D)
    lens = len_ref[...]
    mask_ref[...] = lax.broadcasted_iota(jnp.int32, (BB, L), 1) < lens


def kernel(item_id, cate_id, price, length, item_table, cate_table, W, b,
           pos_table):
    ids_i = item_id.astype(jnp.int32).reshape(NW, NCH, CH)
    ids_c = cate_id.astype(jnp.int32).reshape(NW, NCH, CH)
    gathered = _sc_gather_sum(ids_i, ids_c, item_table, cate_table)
    gathered = gathered.reshape(B, L, DW)

    pos = pos_table[:L][None]
    b2 = b.reshape(1, D)
    seq, mask = pl.pallas_call(
        _tc_body,
        grid=(_GRID,),
        in_specs=[
            pl.BlockSpec((BB, L, DW), lambda i: (i, 0, 0)),
            pl.BlockSpec((BB, L, DN), lambda i: (i, 0, 0)),
            pl.BlockSpec((DN, D), lambda i: (0, 0)),
            pl.BlockSpec((1, D), lambda i: (0, 0)),
            pl.BlockSpec((1, L, D), lambda i: (0, 0, 0)),
            pl.BlockSpec((BB, 1), lambda i: (i, 0)),
        ],
        out_specs=[
            pl.BlockSpec((BB, L, D), lambda i: (i, 0, 0)),
            pl.BlockSpec((BB, L), lambda i: (i, 0)),
        ],
        out_shape=[
            jax.ShapeDtypeStruct((B, L, D), jnp.float32),
            jax.ShapeDtypeStruct((B, L), jnp.bool_),
        ],
    )(gathered, price, W, b2, pos, length)
    return seq, mask


# speedup vs baseline: 7.7005x; 1.0098x over previous
import functools

import jax
import jax.numpy as jnp
from jax import lax
from jax.experimental import pallas as pl
from jax.experimental.pallas import tpu as pltpu
from jax.experimental.pallas import tpu_sc as plsc

B, L, DN, D = 4096, 200, 128, 128
V_ITEM, V_CATE = 100002, 1002
DW = D // 2
BL = B * L
NC, NS = 2, 16
NW = NC * NS
PER_W = BL // NW
CH = 64
NCH = PER_W // CH

_mesh = plsc.VectorSubcoreMesh(core_axis_name="c", subcore_axis_name="s")


@functools.partial(
    pl.kernel,
    out_type=jax.ShapeDtypeStruct((BL, DW), jnp.int32),
    mesh=_mesh,
    compiler_params=pltpu.CompilerParams(needs_layout_passes=False),
    scratch_types=[
        pltpu.VMEM((2, CH), jnp.int32),
        pltpu.VMEM((2, CH), jnp.int32),
        pltpu.VMEM((2, CH, D), jnp.float32),
        pltpu.VMEM((2, CH, D), jnp.float32),
        pltpu.VMEM((CH, DW), jnp.int32),
        pltpu.VMEM_SHARED((V_CATE, D), jnp.float32),
        pltpu.SemaphoreType.DMA,
        pltpu.SemaphoreType.DMA,
        pltpu.SemaphoreType.DMA,
        pltpu.SemaphoreType.DMA,
        pltpu.SemaphoreType.DMA,
        pltpu.SemaphoreType.DMA,
    ],
)
def _sc_gather_sum(item_id_hbm, cate_id_hbm, item_tab_hbm, cate_tab_hbm,
                   out_hbm, idx_i, idx_c, rows_i, rows_c, out_st, cate_sh,
                   sem_i0, sem_i1, sem_c0, sem_c1, sem_x0, sem_x1):
    c = lax.axis_index("c")
    s = lax.axis_index("s")
    wid = s * NC + c
    sem_i = (sem_i0, sem_i1)
    sem_c = (sem_c0, sem_c1)
    sem_x = (sem_x0, sem_x1)

    @pl.when(s == 0)
    def _():
        pltpu.sync_copy(cate_tab_hbm, cate_sh)
    plsc.subcore_barrier()

    base = wid * PER_W

    def issue_idx(j, b):
        pltpu.async_copy(item_id_hbm.at[wid, j], idx_i.at[b], sem_x[b])
        pltpu.async_copy(cate_id_hbm.at[wid, j], idx_c.at[b], sem_x[b])

    def wait_idx(j, b):
        pltpu.make_async_copy(item_id_hbm.at[wid, j], idx_i.at[b],
                              sem_x[b]).wait()
        pltpu.make_async_copy(cate_id_hbm.at[wid, j], idx_c.at[b],
                              sem_x[b]).wait()

    def issue_gather(b):
        pltpu.async_copy(item_tab_hbm.at[idx_i.at[b]], rows_i.at[b], sem_i[b])
        pltpu.async_copy(cate_sh.at[idx_c.at[b]], rows_c.at[b], sem_c[b])

    def wait_gather(b):
        pltpu.make_async_copy(item_tab_hbm.at[idx_i.at[b]], rows_i.at[b],
                              sem_i[b]).wait()
        pltpu.make_async_copy(cate_sh.at[idx_c.at[b]], rows_c.at[b],
                              sem_c[b]).wait()

    def process(j, b):
        def add_row(r, carry):
            for q in range(DW // 16):
                lo = pl.ds(q * 16, 16)
                hi = pl.ds(DW + q * 16, 16)
                a = rows_i[b, r, lo] + rows_c[b, r, lo]
                z = rows_i[b, r, hi] + rows_c[b, r, hi]
                w = plsc.pack(a, z, format=plsc.PackFormat.INTERLEAVED)
                out_st[r, lo] = plsc.bitcast(w, jnp.int32)
            return carry

        lax.fori_loop(0, CH, add_row, 0)
        pltpu.sync_copy(out_st, out_hbm.at[pl.ds(base + j * CH, CH)])

    def half(i, par):
        b = par
        j = 2 * i + par
        wait_idx(j + 1, 1 - b)
        issue_gather(1 - b)
        wait_gather(b)

        @pl.when(j + 2 < NCH)
        def _():
            issue_idx(j + 2, b)

        process(j, b)

    issue_idx(0, 0)
    wait_idx(0, 0)
    issue_gather(0)
    issue_idx(1, 1)

    def body(i, carry):
        half(i, 0)
        half(i, 1)
        return carry

    lax.fori_loop(0, NCH // 2 - 1, body, 0)
    jlast = NCH - 2
    wait_idx(jlast + 1, 1)
    issue_gather(1)
    wait_gather(0)
    process(jlast, 0)
    wait_gather(1)
    process(jlast + 1, 1)


BB = 64
_GRID = B // BB


def _tc_body(gath_ref, price_ref, w_ref, b_ref, pos_ref, len_ref,
             out_ref, mask_ref):
    x = price_ref[...].reshape(BB * L, DN)
    y = jnp.dot(x, w_ref[...], preferred_element_type=jnp.float32)
    y = y.reshape(BB, L, D)
    wu = lax.bitcast_convert_type(gath_ref[...], jnp.uint32)
    g_lo = lax.bitcast_convert_type(wu << 16, jnp.float32)
    g_hi = lax.bitcast_convert_type(wu & jnp.uint32(0xFFFF0000), jnp.float32)
    g = jnp.concatenate([g_lo, g_hi], axis=-1)
    out_ref[...] = y + g + pos_ref[...] + b_ref[...].reshape(1, 1, D)
    lens = len_ref[...]
    mask_ref[...] = lax.broadcasted_iota(jnp.int32, (BB, L), 1) < lens


def kernel(item_id, cate_id, price, length, item_table, cate_table, W, b,
           pos_table):
    ids_i = item_id.astype(jnp.int32).reshape(NW, NCH, CH)
    ids_c = cate_id.astype(jnp.int32).reshape(NW, NCH, CH)
    gathered = _sc_gather_sum(ids_i, ids_c, item_table, cate_table)
    gathered = gathered.reshape(B, L, DW)

    pos = pos_table[:L][None]
    b2 = b.reshape(1, D)
    seq, mask = pl.pallas_call(
        _tc_body,
        grid=(_GRID,),
        in_specs=[
            pl.BlockSpec((BB, L, DW), lambda i: (i, 0, 0)),
            pl.BlockSpec((BB, L, DN), lambda i: (i, 0, 0)),
            pl.BlockSpec((DN, D), lambda i: (0, 0)),
            pl.BlockSpec((1, D), lambda i: (0, 0)),
            pl.BlockSpec((1, L, D), lambda i: (0, 0, 0)),
            pl.BlockSpec((BB, 1), lambda i: (i, 0)),
        ],
        out_specs=[
            pl.BlockSpec((BB, L, D), lambda i: (i, 0, 0)),
            pl.BlockSpec((BB, L), lambda i: (i, 0)),
        ],
        out_shape=[
            jax.ShapeDtypeStruct((B, L, D), jnp.float32),
            jax.ShapeDtypeStruct((B, L), jnp.bool_),
        ],
    )(gathered, price, W, b2, pos, length)
    return seq, mask


# speedup vs baseline: 9.0172x; 1.1710x over previous
import functools

import jax
import jax.numpy as jnp
from jax import lax
from jax.experimental import pallas as pl
from jax.experimental.pallas import tpu as pltpu
from jax.experimental.pallas import tpu_sc as plsc

B, L, DN, D = 4096, 200, 128, 128
V_ITEM, V_CATE = 100002, 1002
DW = D // 2
BL = B * L
NC, NS = 2, 16
NW = NC * NS
PER_W = BL // NW
CH = 128
NCH = PER_W // CH

_mesh = plsc.VectorSubcoreMesh(core_axis_name="c", subcore_axis_name="s")


@functools.partial(
    pl.kernel,
    out_type=jax.ShapeDtypeStruct((BL, DW), jnp.int32),
    mesh=_mesh,
    compiler_params=pltpu.CompilerParams(needs_layout_passes=False),
    scratch_types=[
        pltpu.VMEM((2, CH), jnp.int32),
        pltpu.VMEM((2, CH), jnp.int32),
        pltpu.VMEM((2, CH, D), jnp.float32),
        pltpu.VMEM((2, CH, D), jnp.float32),
        pltpu.VMEM((2, CH, DW), jnp.int32),
        pltpu.VMEM_SHARED((V_CATE, D), jnp.float32),
        pltpu.SemaphoreType.DMA,
        pltpu.SemaphoreType.DMA,
        pltpu.SemaphoreType.DMA,
        pltpu.SemaphoreType.DMA,
        pltpu.SemaphoreType.DMA,
        pltpu.SemaphoreType.DMA,
        pltpu.SemaphoreType.DMA,
        pltpu.SemaphoreType.DMA,
    ],
)
def _sc_gather_sum(item_id_hbm, cate_id_hbm, item_tab_hbm, cate_tab_hbm,
                   out_hbm, idx_i, idx_c, rows_i, rows_c, out_st, cate_sh,
                   sem_i0, sem_i1, sem_c0, sem_c1, sem_x0, sem_x1,
                   sem_o0, sem_o1):
    c = lax.axis_index("c")
    s = lax.axis_index("s")
    wid = s * NC + c
    sem_i = (sem_i0, sem_i1)
    sem_c = (sem_c0, sem_c1)
    sem_x = (sem_x0, sem_x1)
    sem_o = (sem_o0, sem_o1)

    @pl.when(s == 0)
    def _():
        pltpu.sync_copy(cate_tab_hbm, cate_sh)
    plsc.subcore_barrier()

    base = wid * PER_W

    def issue_idx(j, b):
        pltpu.async_copy(item_id_hbm.at[wid, j], idx_i.at[b], sem_x[b])
        pltpu.async_copy(cate_id_hbm.at[wid, j], idx_c.at[b], sem_x[b])

    def wait_idx(j, b):
        pltpu.make_async_copy(item_id_hbm.at[wid, j], idx_i.at[b],
                              sem_x[b]).wait()
        pltpu.make_async_copy(cate_id_hbm.at[wid, j], idx_c.at[b],
                              sem_x[b]).wait()

    def issue_gather(b):
        pltpu.async_copy(item_tab_hbm.at[idx_i.at[b]], rows_i.at[b], sem_i[b])
        pltpu.async_copy(cate_sh.at[idx_c.at[b]], rows_c.at[b], sem_c[b])

    def wait_gather(b):
        pltpu.make_async_copy(item_tab_hbm.at[idx_i.at[b]], rows_i.at[b],
                              sem_i[b]).wait()
        pltpu.make_async_copy(cate_sh.at[idx_c.at[b]], rows_c.at[b],
                              sem_c[b]).wait()

    def wait_out(j, b):
        pltpu.make_async_copy(out_st.at[b],
                              out_hbm.at[pl.ds(base + j * CH, CH)],
                              sem_o[b]).wait()

    def process(j, b):
        @pl.when(j >= 2)
        def _():
            wait_out(j - 2, b)

        def add_row(r, carry):
            for q in range(DW // 16):
                lo = pl.ds(q * 16, 16)
                hi = pl.ds(DW + q * 16, 16)
                a = rows_i[b, r, lo] + rows_c[b, r, lo]
                z = rows_i[b, r, hi] + rows_c[b, r, hi]
                w = plsc.pack(a, z, format=plsc.PackFormat.INTERLEAVED)
                out_st[b, r, lo] = plsc.bitcast(w, jnp.int32)
            return carry

        lax.fori_loop(0, CH, add_row, 0)
        pltpu.async_copy(out_st.at[b], out_hbm.at[pl.ds(base + j * CH, CH)],
                         sem_o[b])

    def half(i, par):
        b = par
        j = 2 * i + par
        wait_idx(j + 1, 1 - b)
        issue_gather(1 - b)
        wait_gather(b)

        @pl.when(j + 2 < NCH)
        def _():
            issue_idx(j + 2, b)

        process(j, b)

    issue_idx(0, 0)
    wait_idx(0, 0)
    issue_gather(0)
    issue_idx(1, 1)

    def body(i, carry):
        half(i, 0)
        half(i, 1)
        return carry

    lax.fori_loop(0, NCH // 2 - 1, body, 0)
    jlast = NCH - 2
    wait_idx(jlast + 1, 1)
    issue_gather(1)
    wait_gather(0)
    process(jlast, 0)
    wait_gather(1)
    process(jlast + 1, 1)
    wait_out(jlast, 0)
    wait_out(jlast + 1, 1)


BB = 64
_GRID = B // BB


def _tc_body(gath_ref, price_ref, w_ref, b_ref, pos_ref, len_ref,
             out_ref, mask_ref):
    x = price_ref[...].reshape(BB * L, DN)
    y = jnp.dot(x, w_ref[...], preferred_element_type=jnp.float32)
    y = y.reshape(BB, L, D)
    wu = lax.bitcast_convert_type(gath_ref[...], jnp.uint32)
    g_lo = lax.bitcast_convert_type(wu << 16, jnp.float32)
    g_hi = lax.bitcast_convert_type(wu & jnp.uint32(0xFFFF0000), jnp.float32)
    g = jnp.concatenate([g_lo, g_hi], axis=-1)
    out_ref[...] = y + g + pos_ref[...] + b_ref[...].reshape(1, 1, D)
    lens = len_ref[...]
    mask_ref[...] = lax.broadcasted_iota(jnp.int32, (BB, L), 1) < lens


def kernel(item_id, cate_id, price, length, item_table, cate_table, W, b,
           pos_table):
    ids_i = item_id.astype(jnp.int32).reshape(NW, NCH, CH)
    ids_c = cate_id.astype(jnp.int32).reshape(NW, NCH, CH)
    gathered = _sc_gather_sum(ids_i, ids_c, item_table, cate_table)
    gathered = gathered.reshape(B, L, DW)

    pos = pos_table[:L][None]
    b2 = b.reshape(1, D)
    seq, mask = pl.pallas_call(
        _tc_body,
        grid=(_GRID,),
        in_specs=[
            pl.BlockSpec((BB, L, DW), lambda i: (i, 0, 0)),
            pl.BlockSpec((BB, L, DN), lambda i: (i, 0, 0)),
            pl.BlockSpec((DN, D), lambda i: (0, 0)),
            pl.BlockSpec((1, D), lambda i: (0, 0)),
            pl.BlockSpec((1, L, D), lambda i: (0, 0, 0)),
            pl.BlockSpec((BB, 1), lambda i: (i, 0)),
        ],
        out_specs=[
            pl.BlockSpec((BB, L, D), lambda i: (i, 0, 0)),
            pl.BlockSpec((BB, L), lambda i: (i, 0)),
        ],
        out_shape=[
            jax.ShapeDtypeStruct((B, L, D), jnp.float32),
            jax.ShapeDtypeStruct((B, L), jnp.bool_),
        ],
    )(gathered, price, W, b2, pos, length)
    return seq, mask


# speedup vs baseline: 9.1424x; 1.0139x over previous
import functools

import jax
import jax.numpy as jnp
from jax import lax
from jax.experimental import pallas as pl
from jax.experimental.pallas import tpu as pltpu
from jax.experimental.pallas import tpu_sc as plsc

B, L, DN, D = 4096, 200, 128, 128
V_ITEM, V_CATE = 100002, 1002
DW = D // 2
BL = B * L
NC, NS = 2, 16
NW = NC * NS
CH = 128
NSPLIT = 2
BH = B // NSPLIT
BLH = BL // NSPLIT

_mesh = plsc.VectorSubcoreMesh(core_axis_name="c", subcore_axis_name="s")


def _make_sc_kernel(n_rows):
    per_w = n_rows // NW
    nch = per_w // CH
    assert per_w % CH == 0 and nch % 2 == 0

    @functools.partial(
        pl.kernel,
        out_type=jax.ShapeDtypeStruct((n_rows, DW), jnp.int32),
        mesh=_mesh,
        compiler_params=pltpu.CompilerParams(needs_layout_passes=False),
        scratch_types=[
            pltpu.VMEM((2, CH), jnp.int32),
            pltpu.VMEM((2, CH), jnp.int32),
            pltpu.VMEM((2, CH, D), jnp.float32),
            pltpu.VMEM((2, CH, D), jnp.float32),
            pltpu.VMEM((2, CH, DW), jnp.int32),
            pltpu.VMEM_SHARED((V_CATE, D), jnp.float32),
            pltpu.SemaphoreType.DMA,
            pltpu.SemaphoreType.DMA,
            pltpu.SemaphoreType.DMA,
            pltpu.SemaphoreType.DMA,
            pltpu.SemaphoreType.DMA,
            pltpu.SemaphoreType.DMA,
            pltpu.SemaphoreType.DMA,
            pltpu.SemaphoreType.DMA,
        ],
    )
    def sc_gather_sum(item_id_hbm, cate_id_hbm, item_tab_hbm, cate_tab_hbm,
                      out_hbm, idx_i, idx_c, rows_i, rows_c, out_st, cate_sh,
                      sem_i0, sem_i1, sem_c0, sem_c1, sem_x0, sem_x1,
                      sem_o0, sem_o1):
        c = lax.axis_index("c")
        s = lax.axis_index("s")
        wid = s * NC + c
        sem_i = (sem_i0, sem_i1)
        sem_c = (sem_c0, sem_c1)
        sem_x = (sem_x0, sem_x1)
        sem_o = (sem_o0, sem_o1)

        @pl.when(s == 0)
        def _():
            pltpu.sync_copy(cate_tab_hbm, cate_sh)
        plsc.subcore_barrier()

        base = wid * per_w

        def issue_idx(j, b):
            pltpu.async_copy(item_id_hbm.at[wid, j], idx_i.at[b], sem_x[b])
            pltpu.async_copy(cate_id_hbm.at[wid, j], idx_c.at[b], sem_x[b])

        def wait_idx(j, b):
            pltpu.make_async_copy(item_id_hbm.at[wid, j], idx_i.at[b],
                                  sem_x[b]).wait()
            pltpu.make_async_copy(cate_id_hbm.at[wid, j], idx_c.at[b],
                                  sem_x[b]).wait()

        def issue_gather(b):
            pltpu.async_copy(item_tab_hbm.at[idx_i.at[b]], rows_i.at[b],
                             sem_i[b])
            pltpu.async_copy(cate_sh.at[idx_c.at[b]], rows_c.at[b], sem_c[b])

        def wait_gather(b):
            pltpu.make_async_copy(item_tab_hbm.at[idx_i.at[b]], rows_i.at[b],
                                  sem_i[b]).wait()
            pltpu.make_async_copy(cate_sh.at[idx_c.at[b]], rows_c.at[b],
                                  sem_c[b]).wait()

        def wait_out(j, b):
            pltpu.make_async_copy(out_st.at[b],
                                  out_hbm.at[pl.ds(base + j * CH, CH)],
                                  sem_o[b]).wait()

        def process(j, b):
            @pl.when(j >= 2)
            def _():
                wait_out(j - 2, b)

            def add_row(r, carry):
                for q in range(DW // 16):
                    lo = pl.ds(q * 16, 16)
                    hi = pl.ds(DW + q * 16, 16)
                    a = rows_i[b, r, lo] + rows_c[b, r, lo]
                    z = rows_i[b, r, hi] + rows_c[b, r, hi]
                    w = plsc.pack(a, z, format=plsc.PackFormat.INTERLEAVED)
                    out_st[b, r, lo] = plsc.bitcast(w, jnp.int32)
                return carry

            lax.fori_loop(0, CH, add_row, 0)
            pltpu.async_copy(out_st.at[b],
                             out_hbm.at[pl.ds(base + j * CH, CH)], sem_o[b])

        def half(i, par):
            b = par
            j = 2 * i + par
            wait_idx(j + 1, 1 - b)
            issue_gather(1 - b)
            wait_gather(b)

            @pl.when(j + 2 < nch)
            def _():
                issue_idx(j + 2, b)

            process(j, b)

        issue_idx(0, 0)
        wait_idx(0, 0)
        issue_gather(0)
        issue_idx(1, 1)

        def body(i, carry):
            half(i, 0)
            half(i, 1)
            return carry

        lax.fori_loop(0, nch // 2 - 1, body, 0)
        jlast = nch - 2
        wait_idx(jlast + 1, 1)
        issue_gather(1)
        wait_gather(0)
        process(jlast, 0)
        wait_gather(1)
        process(jlast + 1, 1)
        wait_out(jlast, 0)
        wait_out(jlast + 1, 1)

    return sc_gather_sum


_sc_half = _make_sc_kernel(BLH)

BB = 64
_GRID_H = BH // BB


def _tc_compute(gath_ref, price_ref, w_ref, b_ref, pos_ref, len_ref,
                out_ref, mask_ref):
    x = price_ref[...].reshape(BB * L, DN)
    y = jnp.dot(x, w_ref[...], preferred_element_type=jnp.float32)
    y = y.reshape(BB, L, D)
    wu = lax.bitcast_convert_type(gath_ref[...], jnp.uint32)
    g_lo = lax.bitcast_convert_type(wu << 16, jnp.float32)
    g_hi = lax.bitcast_convert_type(wu & jnp.uint32(0xFFFF0000), jnp.float32)
    g = jnp.concatenate([g_lo, g_hi], axis=-1)
    out_ref[...] = y + g + pos_ref[...] + b_ref[...].reshape(1, 1, D)
    lens = len_ref[...]
    mask_ref[...] = lax.broadcasted_iota(jnp.int32, (BB, L), 1) < lens


def _tc_body0(gath_ref, price_ref, w_ref, b_ref, pos_ref, len_ref,
              out_ref, mask_ref):
    _tc_compute(gath_ref, price_ref, w_ref, b_ref, pos_ref, len_ref,
                out_ref, mask_ref)


def _tc_body1(seq_in_ref, mask_in_ref, gath_ref, price_ref, w_ref, b_ref,
              pos_ref, len_ref, out_ref, mask_ref):
    del seq_in_ref, mask_in_ref
    _tc_compute(gath_ref, price_ref, w_ref, b_ref, pos_ref, len_ref,
                out_ref, mask_ref)


_OUT_SHAPE = [
    jax.ShapeDtypeStruct((B, L, D), jnp.float32),
    jax.ShapeDtypeStruct((B, L), jnp.bool_),
]


def _common_specs(off):
    return [
        pl.BlockSpec((BB, L, DN), lambda i: (i + off, 0, 0)),
        pl.BlockSpec((DN, D), lambda i: (0, 0)),
        pl.BlockSpec((1, D), lambda i: (0, 0)),
        pl.BlockSpec((1, L, D), lambda i: (0, 0, 0)),
        pl.BlockSpec((BB, 1), lambda i: (i + off, 0)),
    ]


def _out_specs(off):
    return [
        pl.BlockSpec((BB, L, D), lambda i: (i + off, 0, 0)),
        pl.BlockSpec((BB, L), lambda i: (i + off, 0)),
    ]


def kernel(item_id, cate_id, price, length, item_table, cate_table, W, b,
           pos_table):
    item_id = item_id.astype(jnp.int32)
    cate_id = cate_id.astype(jnp.int32)
    per_w = BLH // NW
    nch = per_w // CH

    def half_ids(x, h):
        return x[h * BH:(h + 1) * BH].reshape(NW, nch, CH)

    g0 = _sc_half(half_ids(item_id, 0), half_ids(cate_id, 0),
                  item_table, cate_table).reshape(BH, L, DW)
    g1 = _sc_half(half_ids(item_id, 1), half_ids(cate_id, 1),
                  item_table, cate_table).reshape(BH, L, DW)

    pos = pos_table[:L][None]
    b2 = b.reshape(1, D)

    gspec = pl.BlockSpec((BB, L, DW), lambda i: (i, 0, 0))
    seq0, mask0 = pl.pallas_call(
        _tc_body0,
        grid=(_GRID_H,),
        in_specs=[gspec] + _common_specs(0),
        out_specs=_out_specs(0),
        out_shape=_OUT_SHAPE,
    )(g0, price, W, b2, pos, length)

    seq, mask = pl.pallas_call(
        _tc_body1,
        grid=(_GRID_H,),
        in_specs=[
            pl.BlockSpec((1, 8, D), lambda i: (0, 0, 0)),
            pl.BlockSpec((8, L), lambda i: (0, 0)),
            gspec,
        ] + _common_specs(_GRID_H),
        out_specs=_out_specs(_GRID_H),
        out_shape=_OUT_SHAPE,
        input_output_aliases={0: 0, 1: 1},
    )(seq0, mask0, g1, price, W, b2, pos, length)
    return seq, mask
